# baseline (speedup 1.0000x reference)
import jax
import jax.numpy as jnp
from jax.experimental import pallas as pl
from jax.experimental.pallas import tpu as pltpu

_BM = 128


def _ft_body(x_ref, wt_ref, b_ref, o_ref):
    xb = x_ref[...].astype(jnp.bfloat16)
    o_ref[...] = (
        jnp.dot(xb, wt_ref[...], preferred_element_type=jnp.float32) + b_ref[...]
    )


def kernel(x, W, b):
    B, K = x.shape
    O = W.shape[0]
    wt = W.T.astype(jnp.bfloat16)
    b2 = b.reshape(1, O).astype(jnp.float32)
    return pl.pallas_call(
        _ft_body,
        grid=(B // _BM,),
        in_specs=[
            pl.BlockSpec((_BM, K), lambda i: (i, 0)),
            pl.BlockSpec((K, O), lambda i: (0, 0)),
            pl.BlockSpec((1, O), lambda i: (0, 0)),
        ],
        out_specs=pl.BlockSpec((_BM, O), lambda i: (i, 0)),
        out_shape=jax.ShapeDtypeStruct((B, O), jnp.float32),
        compiler_params=pltpu.CompilerParams(
            dimension_semantics=("parallel",),
            vmem_limit_bytes=60 * 1024 * 1024,
        ),
    )(x, wt, b2)
